# initial kernel scaffold (unmeasured)
import jax
import jax.numpy as jnp
from jax import lax
from jax.experimental import pallas as pl
from jax.experimental.pallas import tpu as pltpu

N_DEV = 4
N_LOC_E = 2


def kernel(x, assign, W1, W2):
    t, d = x.shape
    _, _, f = W1.shape
    assign2d = assign.reshape(t, 1)

    def body(x_ref, a_ref, w1_ref, w2_ref, out_ref,
             xg, ag, pbuf, prec, w1b, w2b,
             sx, rx, sa, ra, sp, rp):
        my = lax.axis_index("i")

        barrier = pltpu.get_barrier_semaphore()
        for off in range(1, N_DEV):
            pl.semaphore_signal(
                barrier, inc=1,
                device_id=((my + off) % N_DEV,),
                device_id_type=pl.DeviceIdType.MESH,
            )
        pl.semaphore_wait(barrier, N_DEV - 1)

        xg[my] = x_ref[:].astype(jnp.bfloat16)
        ag[my] = a_ref[:]
        w1b[:] = w1_ref[:].astype(jnp.bfloat16)
        w2b[:] = w2_ref[:].astype(jnp.bfloat16)

        sends = []
        for off in range(1, N_DEV):
            dst = (my + off) % N_DEV
            cx = pltpu.make_async_remote_copy(
                src_ref=xg.at[my], dst_ref=xg.at[my],
                send_sem=sx.at[off], recv_sem=rx.at[my],
                device_id=(dst,), device_id_type=pl.DeviceIdType.MESH,
            )
            cx.start()
            ca = pltpu.make_async_remote_copy(
                src_ref=ag.at[my], dst_ref=ag.at[my],
                send_sem=sa.at[off], recv_sem=ra.at[my],
                device_id=(dst,), device_id_type=pl.DeviceIdType.MESH,
            )
            ca.start()
            sends += [cx, ca]

        def recv_wait(buf, sems, s):
            rcv = pltpu.make_async_remote_copy(
                src_ref=buf.at[s], dst_ref=buf.at[s],
                send_sem=sems.at[s], recv_sem=sems.at[s],
                device_id=(my,), device_id_type=pl.DeviceIdType.MESH,
            )
            rcv.wait_recv()

        def expert_partial(src):
            xs = xg[src]
            asg = ag[src]
            acc = jnp.zeros((t, d), jnp.float32)
            for le in range(N_LOC_E):
                e = my * N_LOC_E + le
                xm = jnp.where(asg == e, xs, jnp.bfloat16(0.0))
                h = jnp.dot(xm, w1b[le], preferred_element_type=jnp.float32)
                h = jnp.maximum(h, 0.0).astype(jnp.bfloat16)
                acc = acc + jnp.dot(h, w2b[le],
                                    preferred_element_type=jnp.float32)
            return acc

        out_ref[:] = expert_partial(my)

        for off in range(1, N_DEV):
            src = (my + off) % N_DEV
            recv_wait(xg, rx, src)
            recv_wait(ag, ra, src)
            pbuf[src] = expert_partial(src).astype(jnp.bfloat16)
            cp = pltpu.make_async_remote_copy(
                src_ref=pbuf.at[src], dst_ref=prec.at[my],
                send_sem=sp.at[off], recv_sem=rp.at[my],
                device_id=(src,), device_id_type=pl.DeviceIdType.MESH,
            )
            cp.start()
            sends.append(cp)

        for off in range(1, N_DEV):
            s = (my + off) % N_DEV
            recv_wait(prec, rp, s)
            out_ref[:] = out_ref[:] + prec[s].astype(jnp.float32)

        for c in sends:
            c.wait_send()

    return pl.pallas_call(
        body,
        out_shape=jax.ShapeDtypeStruct((t, d), jnp.float32),
        in_specs=[
            pl.BlockSpec(memory_space=pltpu.VMEM),
            pl.BlockSpec(memory_space=pltpu.VMEM),
            pl.BlockSpec(memory_space=pltpu.VMEM),
            pl.BlockSpec(memory_space=pltpu.VMEM),
        ],
        out_specs=pl.BlockSpec(memory_space=pltpu.VMEM),
        scratch_shapes=[
            pltpu.VMEM((N_DEV, t, d), jnp.bfloat16),
            pltpu.VMEM((N_DEV, t, 1), jnp.int32),
            pltpu.VMEM((N_DEV, t, d), jnp.bfloat16),
            pltpu.VMEM((N_DEV, t, d), jnp.bfloat16),
            pltpu.VMEM((N_LOC_E, d, f), jnp.bfloat16),
            pltpu.VMEM((N_LOC_E, f, d), jnp.bfloat16),
            pltpu.SemaphoreType.DMA((N_DEV,)),
            pltpu.SemaphoreType.DMA((N_DEV,)),
            pltpu.SemaphoreType.DMA((N_DEV,)),
            pltpu.SemaphoreType.DMA((N_DEV,)),
            pltpu.SemaphoreType.DMA((N_DEV,)),
            pltpu.SemaphoreType.DMA((N_DEV,)),
        ],
        compiler_params=pltpu.CompilerParams(collective_id=0),
    )(x, assign2d, W1, W2)


# baseline (device time: 175108 ns/iter reference)
import jax
import jax.numpy as jnp
from jax import lax
from jax.experimental import pallas as pl
from jax.experimental.pallas import tpu as pltpu

N_DEV = 4
N_LOC_E = 2
TR = 256


def kernel(x, assign, W1, W2):
    t, d = x.shape
    _, _, f = W1.shape
    xb = x.astype(jnp.bfloat16)
    w1b = W1.astype(jnp.bfloat16)
    w2b = W2.astype(jnp.bfloat16)
    assign2d = assign.reshape(t, 1)

    def body(x_ref, a_ref, w1_ref, w2_ref, out_ref,
             xg, ag, pbuf, prec,
             sx, rx, sa, ra, sp, rp):
        my = lax.axis_index("i")

        barrier = pltpu.get_barrier_semaphore()
        for off in range(1, N_DEV):
            pl.semaphore_signal(
                barrier, inc=1,
                device_id=((my + off) % N_DEV,),
                device_id_type=pl.DeviceIdType.MESH,
            )
        pl.semaphore_wait(barrier, N_DEV - 1)

        xg[my] = x_ref[:]
        ag[my] = a_ref[:]

        sends = []
        for off in range(1, N_DEV):
            dst = (my + off) % N_DEV
            cx = pltpu.make_async_remote_copy(
                src_ref=xg.at[my], dst_ref=xg.at[my],
                send_sem=sx.at[off], recv_sem=rx.at[my],
                device_id=(dst,), device_id_type=pl.DeviceIdType.MESH,
            )
            cx.start()
            ca = pltpu.make_async_remote_copy(
                src_ref=ag.at[my], dst_ref=ag.at[my],
                send_sem=sa.at[off], recv_sem=ra.at[my],
                device_id=(dst,), device_id_type=pl.DeviceIdType.MESH,
            )
            ca.start()
            sends += [cx, ca]

        def recv_wait(buf, sems, s):
            rcv = pltpu.make_async_remote_copy(
                src_ref=buf.at[s], dst_ref=buf.at[s],
                send_sem=sems.at[s], recv_sem=sems.at[s],
                device_id=(my,), device_id_type=pl.DeviceIdType.MESH,
            )
            rcv.wait_recv()

        def compute_chunk(src, pslot):
            def tile(rt, _):
                rows = pl.ds(rt * TR, TR)
                xs = xg[src, rows, :]
                asg = ag[src, rows, :]
                acc = None
                for le in range(N_LOC_E):
                    e = my * N_LOC_E + le
                    xm = jnp.where(asg == e, xs, jnp.bfloat16(0.0))
                    h = jnp.dot(xm, w1_ref[le],
                                preferred_element_type=jnp.float32)
                    h = jnp.maximum(h, 0.0).astype(jnp.bfloat16)
                    p = jnp.dot(h, w2_ref[le],
                                preferred_element_type=jnp.float32)
                    acc = p if acc is None else acc + p
                if pslot is None:
                    out_ref[rows, :] = acc
                else:
                    pbuf[pslot, rows, :] = acc.astype(jnp.bfloat16)
                return 0

            lax.fori_loop(0, t // TR, tile, 0)

        compute_chunk(my, pslot=None)

        for off in range(1, N_DEV):
            src = (my + off) % N_DEV
            recv_wait(xg, rx, src)
            recv_wait(ag, ra, src)
            compute_chunk(src, pslot=off - 1)
            cp = pltpu.make_async_remote_copy(
                src_ref=pbuf.at[off - 1], dst_ref=prec.at[N_DEV - 1 - off],
                send_sem=sp.at[off], recv_sem=rp.at[N_DEV - 1 - off],
                device_id=(src,), device_id_type=pl.DeviceIdType.MESH,
            )
            cp.start()
            sends.append(cp)

        for off in range(1, N_DEV):
            recv_wait(prec, rp, off - 1)
            out_ref[:] = out_ref[:] + prec[off - 1].astype(jnp.float32)

        for c in sends:
            c.wait_send()

    return pl.pallas_call(
        body,
        out_shape=jax.ShapeDtypeStruct((t, d), jnp.float32),
        in_specs=[
            pl.BlockSpec(memory_space=pltpu.VMEM),
            pl.BlockSpec(memory_space=pltpu.VMEM),
            pl.BlockSpec(memory_space=pltpu.VMEM),
            pl.BlockSpec(memory_space=pltpu.VMEM),
        ],
        out_specs=pl.BlockSpec(memory_space=pltpu.VMEM),
        scratch_shapes=[
            pltpu.VMEM((N_DEV, t, d), jnp.bfloat16),
            pltpu.VMEM((N_DEV, t, 1), jnp.int32),
            pltpu.VMEM((N_DEV - 1, t, d), jnp.bfloat16),
            pltpu.VMEM((N_DEV - 1, t, d), jnp.bfloat16),
            pltpu.SemaphoreType.DMA((N_DEV,)),
            pltpu.SemaphoreType.DMA((N_DEV,)),
            pltpu.SemaphoreType.DMA((N_DEV,)),
            pltpu.SemaphoreType.DMA((N_DEV,)),
            pltpu.SemaphoreType.DMA((N_DEV,)),
            pltpu.SemaphoreType.DMA((N_DEV,)),
        ],
        compiler_params=pltpu.CompilerParams(
            collective_id=0,
            vmem_limit_bytes=40 * 1024 * 1024,
        ),
    )(xb, assign2d, w1b, w2b)


# device time: 139198 ns/iter; 1.2580x vs baseline; 1.2580x over previous
import jax
import jax.numpy as jnp
from jax import lax
from jax.experimental import pallas as pl
from jax.experimental.pallas import tpu as pltpu

N_DEV = 4
N_LOC_E = 2
CAP = 384


def kernel(x, assign, W1, W2):
    t, d = x.shape
    _, _, f = W1.shape
    w1b = W1.astype(jnp.bfloat16)
    w2b = W2.astype(jnp.bfloat16)

    dest = assign // N_LOC_E
    order = jnp.argsort(dest, stable=True)
    sd = dest[order]
    starts = jnp.searchsorted(sd, jnp.arange(N_DEV))
    pos = jnp.arange(t) - starts[sd]
    xsend = jnp.zeros((N_DEV, CAP, d), jnp.bfloat16)
    xsend = xsend.at[sd, pos].set(x[order].astype(jnp.bfloat16))
    asend = jnp.full((N_DEV, CAP, 1), -1, jnp.int32)
    asend = asend.at[sd, pos, 0].set(assign[order])

    def body(xs_ref, as_ref, w1_ref, w2_ref, out_ref,
             xr, ar, pbuf,
             sx, rx, sa, ra, sp, rp):
        my = lax.axis_index("i")

        barrier = pltpu.get_barrier_semaphore()
        for off in range(1, N_DEV):
            pl.semaphore_signal(
                barrier, inc=1,
                device_id=((my + off) % N_DEV,),
                device_id_type=pl.DeviceIdType.MESH,
            )
        pl.semaphore_wait(barrier, N_DEV - 1)

        xr[my] = xs_ref[my]
        ar[my] = as_ref[my]

        sends = []
        for off in range(1, N_DEV):
            dst = (my + off) % N_DEV
            cx = pltpu.make_async_remote_copy(
                src_ref=xs_ref.at[dst], dst_ref=xr.at[my],
                send_sem=sx.at[off], recv_sem=rx.at[my],
                device_id=(dst,), device_id_type=pl.DeviceIdType.MESH,
            )
            cx.start()
            ca = pltpu.make_async_remote_copy(
                src_ref=as_ref.at[dst], dst_ref=ar.at[my],
                send_sem=sa.at[off], recv_sem=ra.at[my],
                device_id=(dst,), device_id_type=pl.DeviceIdType.MESH,
            )
            ca.start()
            sends += [cx, ca]

        def recv_wait(buf, sems, s):
            rcv = pltpu.make_async_remote_copy(
                src_ref=buf.at[s], dst_ref=buf.at[s],
                send_sem=sems.at[s], recv_sem=sems.at[s],
                device_id=(my,), device_id_type=pl.DeviceIdType.MESH,
            )
            rcv.wait_recv()

        def bucket_partial(src):
            xs = xr[src]
            asg = ar[src]
            acc = None
            for le in range(N_LOC_E):
                e = my * N_LOC_E + le
                xm = jnp.where(asg == e, xs, jnp.bfloat16(0.0))
                h = jnp.dot(xm, w1_ref[le], preferred_element_type=jnp.float32)
                h = jnp.maximum(h, 0.0).astype(jnp.bfloat16)
                p = jnp.dot(h, w2_ref[le], preferred_element_type=jnp.float32)
                acc = p if acc is None else acc + p
            return acc.astype(jnp.bfloat16)

        out_ref[my] = bucket_partial(my)

        for off in range(1, N_DEV):
            src = (my + off) % N_DEV
            recv_wait(xr, rx, src)
            recv_wait(ar, ra, src)
            pbuf[off - 1] = bucket_partial(src)
            cp = pltpu.make_async_remote_copy(
                src_ref=pbuf.at[off - 1], dst_ref=out_ref.at[my],
                send_sem=sp.at[off], recv_sem=rp.at[my],
                device_id=(src,), device_id_type=pl.DeviceIdType.MESH,
            )
            cp.start()
            sends.append(cp)

        for off in range(1, N_DEV):
            s = (my + off) % N_DEV
            recv_wait(out_ref, rp, s)

        for c in sends:
            c.wait_send()

    outb = pl.pallas_call(
        body,
        out_shape=jax.ShapeDtypeStruct((N_DEV, CAP, d), jnp.bfloat16),
        in_specs=[
            pl.BlockSpec(memory_space=pltpu.VMEM),
            pl.BlockSpec(memory_space=pltpu.VMEM),
            pl.BlockSpec(memory_space=pltpu.VMEM),
            pl.BlockSpec(memory_space=pltpu.VMEM),
        ],
        out_specs=pl.BlockSpec(memory_space=pltpu.VMEM),
        scratch_shapes=[
            pltpu.VMEM((N_DEV, CAP, d), jnp.bfloat16),
            pltpu.VMEM((N_DEV, CAP, 1), jnp.int32),
            pltpu.VMEM((N_DEV - 1, CAP, d), jnp.bfloat16),
            pltpu.SemaphoreType.DMA((N_DEV,)),
            pltpu.SemaphoreType.DMA((N_DEV,)),
            pltpu.SemaphoreType.DMA((N_DEV,)),
            pltpu.SemaphoreType.DMA((N_DEV,)),
            pltpu.SemaphoreType.DMA((N_DEV,)),
            pltpu.SemaphoreType.DMA((N_DEV,)),
        ],
        compiler_params=pltpu.CompilerParams(
            collective_id=0,
            vmem_limit_bytes=40 * 1024 * 1024,
        ),
    )(xsend, asend, w1b, w2b)

    y = outb[sd, pos].astype(jnp.float32)
    return jnp.zeros((t, d), jnp.float32).at[order].set(y)
